# baseline (speedup 1.0000x reference)
import functools

import jax
import jax.numpy as jnp
from jax import lax
from jax.experimental import pallas as pl
from jax.experimental.pallas import tpu as pltpu
from jax.experimental.pallas import tpu_sc as plsc

N = 10000
E = 320000
G = 64
D = 256

NC = 2
NS = 16
NW = NC * NS
R = 320
RP = R + 16
P = NW * R
SUB = 10
CHUNK = 1600
CAP = 1280
B = 16
NT = 1000
ET = 4000


def _node_proj_body(x_ref, wpre_ref, bpre_ref, wl_ref, bl_ref, wr_ref, br_ref,
                    xl_ref, xr_ref):
    h = jnp.maximum(
        jnp.dot(x_ref[...], wpre_ref[...], preferred_element_type=jnp.float32)
        + bpre_ref[...], 0.0)
    xl_ref[...] = jnp.dot(h, wl_ref[...], preferred_element_type=jnp.float32) + bl_ref[...]
    xr_ref[...] = jnp.dot(h, wr_ref[...], preferred_element_type=jnp.float32) + br_ref[...]


def _edge_proj_body(ea_ref, wpe_ref, bpe_ref, we_ref, ep_ref):
    ef = jnp.maximum(
        jnp.dot(ea_ref[...], wpe_ref[...], preferred_element_type=jnp.float32)
        + bpe_ref[...], 0.0)
    ep_ref[...] = jnp.dot(ef, we_ref[...], preferred_element_type=jnp.float32)


def _pool_body(out_ref, den_ref, batch_ref, cb_ref, wp_ref, bp_ref,
               xpool_ref, logits_ref, probs_ref):
    den = den_ref[...][:, 0:1]
    out = jnp.maximum(out_ref[...] / (den + 1e-16) + cb_ref[...], 0.0)
    b = batch_ref[...]
    gids = lax.broadcasted_iota(jnp.int32, (G, P), 0)
    onehot = (gids == b).astype(jnp.float32)
    sums = jnp.dot(onehot, out, preferred_element_type=jnp.float32)
    counts = jnp.sum(onehot, axis=1, keepdims=True)
    xpool = sums / jnp.maximum(counts, 1.0)
    xpool_ref[...] = xpool
    logits = jnp.dot(xpool, wp_ref[...], preferred_element_type=jnp.float32) + bp_ref[...]
    logits_ref[...] = logits
    z = logits - jnp.max(logits, axis=1, keepdims=True)
    ez = jnp.exp(z)
    probs_ref[...] = ez / jnp.sum(ez, axis=1, keepdims=True)


def _scan_body(src_hbm, dst_hbm,
               lsrc_hbm, ldst_hbm, leid_hbm, cnts_hbm,
               src_l, dstg_l, eid_l, stage_s0, stage_s1, stage_d0, stage_d1,
               cnt_v, sem3, sem4, semo):
    c = lax.axis_index("c")
    s = lax.axis_index("s")
    w = c * NS + s
    lo = w * R
    stages = ((stage_s0, stage_d0), (stage_s1, stage_d1))
    lanes16 = lax.iota(jnp.int32, 16)
    nch = (E // SUB) // CHUNK

    def scan_buf(buf_sel, cbase, cnt):
        st_s, st_d = stages[buf_sel]

        def scan_step(i, cnt):
            sv = st_s[pl.ds(i * 16, 16)]
            dv = st_d[pl.ds(i * 16, 16)]
            msk = (dv >= lo) & (dv < lo + R)
            npc = plsc.all_reduce_population_count(msk)[0]

            @pl.when(npc > 0)
            def _store():
                x = msk.astype(jnp.int32)
                for k in (1, 2, 4, 8):
                    sh = x.at[jnp.maximum(lanes16 - k, 0)].get(
                        mode="promise_in_bounds")
                    x = jnp.where(lanes16 >= k, x + sh, x)
                pos = cnt + x - 1
                eidv = cbase + i * 16 + lanes16
                plsc.store_scatter(src_l, [pos], sv, mask=msk)
                plsc.store_scatter(dstg_l, [pos], dv, mask=msk)
                plsc.store_scatter(eid_l, [pos], eidv, mask=msk)
            return cnt + npc
        return lax.fori_loop(0, CHUNK // 16, scan_step, cnt)

    def do_subslice(u, _):
        ubase = u * (E // SUB)
        pltpu.async_copy(src_hbm.at[pl.ds(ubase, CHUNK)], stage_s0, sem3)
        pltpu.async_copy(dst_hbm.at[pl.ds(ubase, CHUNK)], stage_d0, sem4)

        def chunk_pair(g, cnt):
            for b in range(2):
                ch = g * 2 + b
                cbase = ubase + ch * CHUNK
                pltpu.make_async_copy(src_hbm.at[pl.ds(cbase, CHUNK)],
                                      stages[b][0], sem3).wait()
                pltpu.make_async_copy(dst_hbm.at[pl.ds(cbase, CHUNK)],
                                      stages[b][1], sem4).wait()

                @pl.when(ch + 1 < nch)
                def _prefetch(cbase=cbase, b=b):
                    nxt = cbase + CHUNK
                    pltpu.async_copy(src_hbm.at[pl.ds(nxt, CHUNK)],
                                     stages[1 - b][0], sem3)
                    pltpu.async_copy(dst_hbm.at[pl.ds(nxt, CHUNK)],
                                     stages[1 - b][1], sem4)
                cnt = scan_buf(b, cbase, cnt)
            return cnt

        cnt = lax.fori_loop(0, nch // 2, chunk_pair, jnp.int32(0))

        pad_src = jnp.zeros((16,), jnp.int32)
        pad_dst = jnp.full((16,), lo + R, jnp.int32)
        for k in range(B // 16):
            src_l[pl.ds(cnt + k * 16, 16)] = pad_src
            dstg_l[pl.ds(cnt + k * 16, 16)] = pad_dst
            eid_l[pl.ds(cnt + k * 16, 16)] = pad_src
        cnt_v[pl.ds(u * 16, 16)] = jnp.broadcast_to(cnt, (16,))
        cp0 = pltpu.async_copy(src_l, lsrc_hbm.at[w, u], semo)
        cp1 = pltpu.async_copy(dstg_l, ldst_hbm.at[w, u], semo)
        cp2 = pltpu.async_copy(eid_l, leid_hbm.at[w, u], semo)
        cp0.wait()
        cp1.wait()
        cp2.wait()
        return 0

    lax.fori_loop(0, SUB, do_subslice, 0)
    pltpu.sync_copy(cnt_v, cnts_hbm.at[w])


def _scan_call(src, dst):
    mesh = plsc.VectorSubcoreMesh(core_axis_name="c", subcore_axis_name="s")
    f = pl.kernel(
        _scan_body,
        out_type=[
            jax.ShapeDtypeStruct((NW, SUB, CAP), jnp.int32),
            jax.ShapeDtypeStruct((NW, SUB, CAP), jnp.int32),
            jax.ShapeDtypeStruct((NW, SUB, CAP), jnp.int32),
            jax.ShapeDtypeStruct((NW, SUB * 16), jnp.int32),
        ],
        mesh=mesh,
        compiler_params=pltpu.CompilerParams(needs_layout_passes=False),
        scratch_types=[
            pltpu.VMEM((CAP,), jnp.int32),
            pltpu.VMEM((CAP,), jnp.int32),
            pltpu.VMEM((CAP,), jnp.int32),
            pltpu.VMEM((CHUNK,), jnp.int32),
            pltpu.VMEM((CHUNK,), jnp.int32),
            pltpu.VMEM((CHUNK,), jnp.int32),
            pltpu.VMEM((CHUNK,), jnp.int32),
            pltpu.VMEM((SUB * 16,), jnp.int32),
            pltpu.SemaphoreType.DMA,
            pltpu.SemaphoreType.DMA,
            pltpu.SemaphoreType.DMA,
        ],
    )
    return f(src, dst)


def _sc_body(lsrc_hbm, ldst_hbm, leid_hbm, cnts_hbm,
             xl_hbm, xr_hbm, ep_hbm, att_hbm,
             outs_hbm, dens_hbm,
             src_l, dstg_l, eid_l,
             xl_b0, xr_b0, ep_b0, xl_b1, xr_b1, ep_b1,
             sbk0, dbk0, ebk0, sbk1, dbk1, ebk1,
             att_v, cnt_v, acc, den_acc, dstl_smem, cnt_smem,
             gs0, gs1, gs2, gs3, gs4, gs5, seml):
    c = lax.axis_index("c")
    s = lax.axis_index("s")
    w = c * NS + s
    lo = w * R
    zero16 = jnp.zeros((16,), jnp.float32)
    bufs = ((xl_b0, xr_b0, ep_b0, sbk0, dbk0, ebk0, (gs0, gs1, gs2)),
            (xl_b1, xr_b1, ep_b1, sbk1, dbk1, ebk1, (gs3, gs4, gs5)))

    lanes16 = lax.iota(jnp.int32, 16)

    def zero_rows(i, _):
        for j in range(16):
            acc[pl.ds(i * 256 + j * 16, 16)] = zero16
        den_acc[pl.ds(i * 16, 16)] = zero16
        return 0
    lax.fori_loop(0, RP, zero_rows, 0)
    pltpu.sync_copy(att_hbm, att_v)
    pltpu.sync_copy(cnts_hbm.at[w], cnt_v)
    for u in range(SUB):
        cv = cnt_v[pl.ds(u * 16, 16)]
        cnt_smem[u] = cv[0]

    att_regs = [att_v[pl.ds(j * 16, 16)] for j in range(16)]

    def prep_block(blk, bs):
        xl_buf, xr_buf, ep_buf, sbk, dbk, ebk, sems = bufs[bs]
        base = blk * B
        dg = dstg_l[pl.ds(base, 16)]
        sbk[:] = src_l[pl.ds(base, 16)]
        dbk[:] = dg
        ebk[:] = eid_l[pl.ds(base, 16)]
        dl = dg - lo
        for e in range(B):
            dstl_smem[bs * B + e] = dl[e]
        pltpu.async_copy(xl_hbm.at[sbk], xl_buf, sems[0])
        pltpu.async_copy(xr_hbm.at[dbk], xr_buf, sems[1])
        pltpu.async_copy(ep_hbm.at[ebk], ep_buf, sems[2])

    def compute_block(bs):
        xl_buf, xr_buf, ep_buf, sbk, dbk, ebk, sems = bufs[bs]
        pltpu.make_async_copy(xl_hbm.at[sbk], xl_buf, sems[0]).wait()
        pltpu.make_async_copy(xr_hbm.at[dbk], xr_buf, sems[1]).wait()
        pltpu.make_async_copy(ep_hbm.at[ebk], ep_buf, sems[2]).wait()

        def edge_step(e2, _):
            for half in range(2):
                e = e2 * 2 + half
                a0 = zero16
                a1 = zero16
                a2 = zero16
                a3 = zero16
                accs = [a0, a1, a2, a3]
                xls = []
                for j in range(16):
                    sl = pl.ds(j * 16, 16)
                    xlj = xl_buf[e, sl]
                    xls.append(xlj)
                    m = xlj + xr_buf[e, sl] + ep_buf[e, sl]
                    m = jnp.maximum(m, 0.2 * m)
                    accs[j % 4] = accs[j % 4] + m * att_regs[j]
                logit = jnp.sum((accs[0] + accs[1]) + (accs[2] + accs[3]))
                wv = jnp.exp(jnp.broadcast_to(logit, (16,)))
                row = dstl_smem[bs * B + e]
                plsc.addupdate_scatter(
                    den_acc, [jnp.broadcast_to(row * 16, (16,)) + lanes16], wv)
                rb = jnp.broadcast_to(row * 256, (16,)) + lanes16
                for j in range(16):
                    plsc.addupdate_scatter(acc, [rb + j * 16], xls[j] * wv)
            return 0
        lax.fori_loop(0, B // 2, edge_step, 0)

    def do_subslice(u, _):
        cl0 = pltpu.async_copy(lsrc_hbm.at[w, u], src_l, seml)
        cl1 = pltpu.async_copy(ldst_hbm.at[w, u], dstg_l, seml)
        cl2 = pltpu.async_copy(leid_hbm.at[w, u], eid_l, seml)
        cl0.wait()
        cl1.wait()
        cl2.wait()
        cnt = cnt_smem[u]
        nblk = (cnt + B - 1) // B

        @pl.when(nblk > 0)
        def _prime():
            prep_block(0, 0)

        def pair(g, _):
            for b2 in range(2):
                blk = g * 2 + b2

                @pl.when(blk < nblk)
                def _do(blk=blk, b2=b2):
                    @pl.when(blk + 1 < nblk)
                    def _pf():
                        prep_block(blk + 1, 1 - b2)
                    compute_block(b2)
            return 0
        lax.fori_loop(0, (nblk + 1) // 2, pair, 0)
        return 0

    lax.fori_loop(0, SUB, do_subslice, 0)

    pltpu.sync_copy(acc.at[pl.ds(0, R * 256)], outs_hbm.at[pl.ds(lo * 256, R * 256)])
    pltpu.sync_copy(den_acc.at[pl.ds(0, R * 16)], dens_hbm.at[pl.ds(lo * 16, R * 16)])


def _sc_call(lists, xl, xr, ep, att):
    lsrc, ldst, leid, cnts = lists
    mesh = plsc.VectorSubcoreMesh(core_axis_name="c", subcore_axis_name="s")
    f = pl.kernel(
        _sc_body,
        out_type=[
            jax.ShapeDtypeStruct((P * D,), jnp.float32),
            jax.ShapeDtypeStruct((P * 16,), jnp.float32),
        ],
        mesh=mesh,
        compiler_params=pltpu.CompilerParams(needs_layout_passes=False),
        scratch_types=[
            pltpu.VMEM((CAP,), jnp.int32),
            pltpu.VMEM((CAP,), jnp.int32),
            pltpu.VMEM((CAP,), jnp.int32),
            pltpu.VMEM((B, D), jnp.float32),
            pltpu.VMEM((B, D), jnp.float32),
            pltpu.VMEM((B, D), jnp.float32),
            pltpu.VMEM((B, D), jnp.float32),
            pltpu.VMEM((B, D), jnp.float32),
            pltpu.VMEM((B, D), jnp.float32),
            pltpu.VMEM((B,), jnp.int32),
            pltpu.VMEM((B,), jnp.int32),
            pltpu.VMEM((B,), jnp.int32),
            pltpu.VMEM((B,), jnp.int32),
            pltpu.VMEM((B,), jnp.int32),
            pltpu.VMEM((B,), jnp.int32),
            pltpu.VMEM((D,), jnp.float32),
            pltpu.VMEM((SUB * 16,), jnp.int32),
            pltpu.VMEM((RP * D,), jnp.float32),
            pltpu.VMEM((RP * 16,), jnp.float32),
            pltpu.SMEM((2 * B,), jnp.int32),
            pltpu.SMEM((SUB,), jnp.int32),
            pltpu.SemaphoreType.DMA,
            pltpu.SemaphoreType.DMA,
            pltpu.SemaphoreType.DMA,
            pltpu.SemaphoreType.DMA,
            pltpu.SemaphoreType.DMA,
            pltpu.SemaphoreType.DMA,
            pltpu.SemaphoreType.DMA,
        ],
    )
    return f(lsrc, ldst, leid, cnts, xl, xr, ep, att)


def kernel(x, edge_index, edge_attr, batch, W_pre_x, b_pre_x, W_pre_e, b_pre_e,
           Wl, bl, Wr, br, We, att, conv_b, W_pred, b_pred):
    xl, xr = pl.pallas_call(
        _node_proj_body,
        grid=(N // NT,),
        in_specs=[
            pl.BlockSpec((NT, 128), lambda i: (i, 0)),
            pl.BlockSpec((128, D), lambda i: (0, 0)),
            pl.BlockSpec((D,), lambda i: (0,)),
            pl.BlockSpec((D, D), lambda i: (0, 0)),
            pl.BlockSpec((D,), lambda i: (0,)),
            pl.BlockSpec((D, D), lambda i: (0, 0)),
            pl.BlockSpec((D,), lambda i: (0,)),
        ],
        out_specs=[
            pl.BlockSpec((NT, D), lambda i: (i, 0)),
            pl.BlockSpec((NT, D), lambda i: (i, 0)),
        ],
        out_shape=[
            jax.ShapeDtypeStruct((N, D), jnp.float32),
            jax.ShapeDtypeStruct((N, D), jnp.float32),
        ],
    )(x, W_pre_x, b_pre_x, Wl, bl, Wr, br)

    ep = pl.pallas_call(
        _edge_proj_body,
        grid=(E // ET,),
        in_specs=[
            pl.BlockSpec((ET, 16), lambda i: (i, 0)),
            pl.BlockSpec((16, 512), lambda i: (0, 0)),
            pl.BlockSpec((512,), lambda i: (0,)),
            pl.BlockSpec((512, D), lambda i: (0, 0)),
        ],
        out_specs=pl.BlockSpec((ET, D), lambda i: (i, 0)),
        out_shape=jax.ShapeDtypeStruct((E, D), jnp.float32),
    )(edge_attr, W_pre_e, b_pre_e, We)

    src = edge_index[0]
    dst = edge_index[1]
    lists = _scan_call(src, dst)
    outs, dens = _sc_call(lists, xl, xr, ep, att)

    out_pad = outs.reshape(P, D)
    den_pad = dens.reshape(P, 16)
    sent = jnp.full((P - N,), G, jnp.int32)
    batch_pad = jnp.concatenate([batch, sent]).reshape(1, P)

    x_pool, logits, probs = pl.pallas_call(
        _pool_body,
        in_specs=[
            pl.BlockSpec((P, D), lambda: (0, 0)),
            pl.BlockSpec((P, 16), lambda: (0, 0)),
            pl.BlockSpec((1, P), lambda: (0, 0)),
            pl.BlockSpec((D,), lambda: (0,)),
            pl.BlockSpec((D, 10), lambda: (0, 0)),
            pl.BlockSpec((10,), lambda: (0,)),
        ],
        out_specs=[
            pl.BlockSpec((G, D), lambda: (0, 0)),
            pl.BlockSpec((G, 10), lambda: (0, 0)),
            pl.BlockSpec((G, 10), lambda: (0, 0)),
        ],
        out_shape=[
            jax.ShapeDtypeStruct((G, D), jnp.float32),
            jax.ShapeDtypeStruct((G, 10), jnp.float32),
            jax.ShapeDtypeStruct((G, 10), jnp.float32),
        ],
    )(out_pad, den_pad, batch_pad, conv_b, W_pred, b_pred)
    return (x_pool, logits, probs)


# speedup vs baseline: 1.0633x; 1.0633x over previous
import functools

import jax
import jax.numpy as jnp
from jax import lax
from jax.experimental import pallas as pl
from jax.experimental.pallas import tpu as pltpu
from jax.experimental.pallas import tpu_sc as plsc

N = 10000
E = 320000
G = 64
D = 256

NC = 2
NS = 16
NW = NC * NS
R = 320
RP = R + 16
P = NW * R
SUB = 10
CHUNK = 1600
CAP = 1280
B = 16
NT = 1000
ET = 4000


def _node_proj_body(x_ref, wpre_ref, bpre_ref, wl_ref, bl_ref, wr_ref, br_ref,
                    xl_ref, xr_ref):
    h = jnp.maximum(
        jnp.dot(x_ref[...], wpre_ref[...], preferred_element_type=jnp.float32)
        + bpre_ref[...], 0.0)
    xl_ref[...] = jnp.dot(h, wl_ref[...], preferred_element_type=jnp.float32) + bl_ref[...]
    xr_ref[...] = jnp.dot(h, wr_ref[...], preferred_element_type=jnp.float32) + br_ref[...]


def _edge_proj_body(ea_ref, wpe_ref, bpe_ref, we_ref, ep_ref):
    ef = jnp.maximum(
        jnp.dot(ea_ref[...], wpe_ref[...], preferred_element_type=jnp.float32)
        + bpe_ref[...], 0.0)
    ep_ref[...] = jnp.dot(ef, we_ref[...], preferred_element_type=jnp.float32)


def _pool_body(out_ref, den_ref, batch_ref, cb_ref, wp_ref, bp_ref,
               xpool_ref, logits_ref, probs_ref):
    den = den_ref[...][:, 0:1]
    out = jnp.maximum(out_ref[...] / (den + 1e-16) + cb_ref[...], 0.0)
    b = batch_ref[...]
    gids = lax.broadcasted_iota(jnp.int32, (G, P), 0)
    onehot = (gids == b).astype(jnp.float32)
    sums = jnp.dot(onehot, out, preferred_element_type=jnp.float32)
    counts = jnp.sum(onehot, axis=1, keepdims=True)
    xpool = sums / jnp.maximum(counts, 1.0)
    xpool_ref[...] = xpool
    logits = jnp.dot(xpool, wp_ref[...], preferred_element_type=jnp.float32) + bp_ref[...]
    logits_ref[...] = logits
    z = logits - jnp.max(logits, axis=1, keepdims=True)
    ez = jnp.exp(z)
    probs_ref[...] = ez / jnp.sum(ez, axis=1, keepdims=True)


def _scan_body(src_hbm, dst_hbm,
               lsrc_hbm, ldst_hbm, leid_hbm, cnts_hbm,
               src_l, dstg_l, eid_l, stage_s0, stage_s1, stage_d0, stage_d1,
               cnt_v, sem3, sem4, semo):
    c = lax.axis_index("c")
    s = lax.axis_index("s")
    w = c * NS + s
    lo = w * R
    stages = ((stage_s0, stage_d0), (stage_s1, stage_d1))
    lanes16 = lax.iota(jnp.int32, 16)
    nch = (E // SUB) // CHUNK

    def scan_buf(buf_sel, cbase, cnt):
        st_s, st_d = stages[buf_sel]

        def scan_step(q, cnt):
            for h in range(4):
                i = q * 4 + h
                sv = st_s[pl.ds(i * 16, 16)]
                dv = st_d[pl.ds(i * 16, 16)]
                msk = (dv >= lo) & (dv < lo + R)
                npc = plsc.all_reduce_population_count(msk)[0]

                @pl.when(npc > 0)
                def _store(i=i, cnt=cnt, sv=sv, dv=dv, msk=msk):
                    x = msk.astype(jnp.int32)
                    for k in (1, 2, 4, 8):
                        sh = x.at[jnp.maximum(lanes16 - k, 0)].get(
                            mode="promise_in_bounds")
                        x = jnp.where(lanes16 >= k, x + sh, x)
                    pos = cnt + x - 1
                    eidv = cbase + i * 16 + lanes16
                    plsc.store_scatter(src_l, [pos], sv, mask=msk)
                    plsc.store_scatter(dstg_l, [pos], dv, mask=msk)
                    plsc.store_scatter(eid_l, [pos], eidv, mask=msk)
                cnt = cnt + npc
            return cnt
        return lax.fori_loop(0, CHUNK // 64, scan_step, cnt)

    def do_subslice(u, _):
        ubase = u * (E // SUB)
        pltpu.async_copy(src_hbm.at[pl.ds(ubase, CHUNK)], stage_s0, sem3)
        pltpu.async_copy(dst_hbm.at[pl.ds(ubase, CHUNK)], stage_d0, sem4)

        def chunk_pair(g, cnt):
            for b in range(2):
                ch = g * 2 + b
                cbase = ubase + ch * CHUNK
                pltpu.make_async_copy(src_hbm.at[pl.ds(cbase, CHUNK)],
                                      stages[b][0], sem3).wait()
                pltpu.make_async_copy(dst_hbm.at[pl.ds(cbase, CHUNK)],
                                      stages[b][1], sem4).wait()

                @pl.when(ch + 1 < nch)
                def _prefetch(cbase=cbase, b=b):
                    nxt = cbase + CHUNK
                    pltpu.async_copy(src_hbm.at[pl.ds(nxt, CHUNK)],
                                     stages[1 - b][0], sem3)
                    pltpu.async_copy(dst_hbm.at[pl.ds(nxt, CHUNK)],
                                     stages[1 - b][1], sem4)
                cnt = scan_buf(b, cbase, cnt)
            return cnt

        cnt = lax.fori_loop(0, nch // 2, chunk_pair, jnp.int32(0))

        pad_src = jnp.zeros((16,), jnp.int32)
        pad_dst = jnp.full((16,), lo + R, jnp.int32)
        for k in range(B // 16):
            src_l[pl.ds(cnt + k * 16, 16)] = pad_src
            dstg_l[pl.ds(cnt + k * 16, 16)] = pad_dst
            eid_l[pl.ds(cnt + k * 16, 16)] = pad_src
        cnt_v[pl.ds(u * 16, 16)] = jnp.broadcast_to(cnt, (16,))
        cp0 = pltpu.async_copy(src_l, lsrc_hbm.at[w, u], semo)
        cp1 = pltpu.async_copy(dstg_l, ldst_hbm.at[w, u], semo)
        cp2 = pltpu.async_copy(eid_l, leid_hbm.at[w, u], semo)
        cp0.wait()
        cp1.wait()
        cp2.wait()
        return 0

    lax.fori_loop(0, SUB, do_subslice, 0)
    pltpu.sync_copy(cnt_v, cnts_hbm.at[w])


def _scan_call(src, dst):
    mesh = plsc.VectorSubcoreMesh(core_axis_name="c", subcore_axis_name="s")
    f = pl.kernel(
        _scan_body,
        out_type=[
            jax.ShapeDtypeStruct((NW, SUB, CAP), jnp.int32),
            jax.ShapeDtypeStruct((NW, SUB, CAP), jnp.int32),
            jax.ShapeDtypeStruct((NW, SUB, CAP), jnp.int32),
            jax.ShapeDtypeStruct((NW, SUB * 16), jnp.int32),
        ],
        mesh=mesh,
        compiler_params=pltpu.CompilerParams(needs_layout_passes=False),
        scratch_types=[
            pltpu.VMEM((CAP,), jnp.int32),
            pltpu.VMEM((CAP,), jnp.int32),
            pltpu.VMEM((CAP,), jnp.int32),
            pltpu.VMEM((CHUNK,), jnp.int32),
            pltpu.VMEM((CHUNK,), jnp.int32),
            pltpu.VMEM((CHUNK,), jnp.int32),
            pltpu.VMEM((CHUNK,), jnp.int32),
            pltpu.VMEM((SUB * 16,), jnp.int32),
            pltpu.SemaphoreType.DMA,
            pltpu.SemaphoreType.DMA,
            pltpu.SemaphoreType.DMA,
        ],
    )
    return f(src, dst)


def _sc_body(lsrc_hbm, ldst_hbm, leid_hbm, cnts_hbm,
             xl_hbm, xr_hbm, ep_hbm, att_hbm,
             outs_hbm, dens_hbm,
             src_l, dstg_l, eid_l,
             xl_b0, xr_b0, ep_b0, xl_b1, xr_b1, ep_b1,
             sbk0, dbk0, ebk0, sbk1, dbk1, ebk1,
             att_v, cnt_v, acc, den_acc, dstl_smem, cnt_smem,
             gs0, gs1, gs2, gs3, gs4, gs5, seml):
    c = lax.axis_index("c")
    s = lax.axis_index("s")
    w = c * NS + s
    lo = w * R
    zero16 = jnp.zeros((16,), jnp.float32)
    bufs = ((xl_b0, xr_b0, ep_b0, sbk0, dbk0, ebk0, (gs0, gs1, gs2)),
            (xl_b1, xr_b1, ep_b1, sbk1, dbk1, ebk1, (gs3, gs4, gs5)))

    def zero_rows(i, _):
        for j in range(16):
            acc[i, pl.ds(j * 16, 16)] = zero16
        den_acc[pl.ds(i * 16, 16)] = zero16
        return 0
    lax.fori_loop(0, RP, zero_rows, 0)
    pltpu.sync_copy(att_hbm, att_v)
    pltpu.sync_copy(cnts_hbm.at[w], cnt_v)
    for u in range(SUB):
        cv = cnt_v[pl.ds(u * 16, 16)]
        cnt_smem[u] = cv[0]

    att_regs = [att_v[pl.ds(j * 16, 16)] for j in range(16)]

    def prep_block(blk, bs):
        xl_buf, xr_buf, ep_buf, sbk, dbk, ebk, sems = bufs[bs]
        base = blk * B
        dg = dstg_l[pl.ds(base, 16)]
        sbk[:] = src_l[pl.ds(base, 16)]
        dbk[:] = dg
        ebk[:] = eid_l[pl.ds(base, 16)]
        dl = dg - lo
        for e in range(B):
            dstl_smem[bs * B + e] = dl[e]
        pltpu.async_copy(xl_hbm.at[sbk], xl_buf, sems[0])
        pltpu.async_copy(xr_hbm.at[dbk], xr_buf, sems[1])
        pltpu.async_copy(ep_hbm.at[ebk], ep_buf, sems[2])

    def compute_block(bs):
        xl_buf, xr_buf, ep_buf, sbk, dbk, ebk, sems = bufs[bs]
        pltpu.make_async_copy(xl_hbm.at[sbk], xl_buf, sems[0]).wait()
        pltpu.make_async_copy(xr_hbm.at[dbk], xr_buf, sems[1]).wait()
        pltpu.make_async_copy(ep_hbm.at[ebk], ep_buf, sems[2]).wait()

        def edge_step(e2, _):
            for half in range(2):
                e = e2 * 2 + half
                a0 = zero16
                a1 = zero16
                a2 = zero16
                a3 = zero16
                accs = [a0, a1, a2, a3]
                xls = []
                for j in range(16):
                    sl = pl.ds(j * 16, 16)
                    xlj = xl_buf[e, sl]
                    xls.append(xlj)
                    m = xlj + xr_buf[e, sl] + ep_buf[e, sl]
                    m = jnp.maximum(m, 0.2 * m)
                    accs[j % 4] = accs[j % 4] + m * att_regs[j]
                logit = jnp.sum((accs[0] + accs[1]) + (accs[2] + accs[3]))
                wv = jnp.exp(jnp.broadcast_to(logit, (16,)))
                row = dstl_smem[bs * B + e]
                plsc.addupdate(den_acc.at[pl.ds(row * 16, 16)], wv)
                for j in range(16):
                    sl = pl.ds(j * 16, 16)
                    plsc.addupdate(acc.at[row, sl], xls[j] * wv)
            return 0
        lax.fori_loop(0, B // 2, edge_step, 0)

    def do_subslice(u, _):
        cl0 = pltpu.async_copy(lsrc_hbm.at[w, u], src_l, seml)
        cl1 = pltpu.async_copy(ldst_hbm.at[w, u], dstg_l, seml)
        cl2 = pltpu.async_copy(leid_hbm.at[w, u], eid_l, seml)
        cl0.wait()
        cl1.wait()
        cl2.wait()
        cnt = cnt_smem[u]
        nblk = (cnt + B - 1) // B

        @pl.when(nblk > 0)
        def _prime():
            prep_block(0, 0)

        def pair(g, _):
            for b2 in range(2):
                blk = g * 2 + b2

                @pl.when(blk < nblk)
                def _do(blk=blk, b2=b2):
                    @pl.when(blk + 1 < nblk)
                    def _pf():
                        prep_block(blk + 1, 1 - b2)
                    compute_block(b2)
            return 0
        lax.fori_loop(0, (nblk + 1) // 2, pair, 0)
        return 0

    lax.fori_loop(0, SUB, do_subslice, 0)

    pltpu.sync_copy(acc.at[pl.ds(0, R)], outs_hbm.at[pl.ds(lo, R)])
    pltpu.sync_copy(den_acc.at[pl.ds(0, R * 16)], dens_hbm.at[pl.ds(lo * 16, R * 16)])


def _sc_call(lists, xl, xr, ep, att):
    lsrc, ldst, leid, cnts = lists
    mesh = plsc.VectorSubcoreMesh(core_axis_name="c", subcore_axis_name="s")
    f = pl.kernel(
        _sc_body,
        out_type=[
            jax.ShapeDtypeStruct((P, D), jnp.float32),
            jax.ShapeDtypeStruct((P * 16,), jnp.float32),
        ],
        mesh=mesh,
        compiler_params=pltpu.CompilerParams(needs_layout_passes=False),
        scratch_types=[
            pltpu.VMEM((CAP,), jnp.int32),
            pltpu.VMEM((CAP,), jnp.int32),
            pltpu.VMEM((CAP,), jnp.int32),
            pltpu.VMEM((B, D), jnp.float32),
            pltpu.VMEM((B, D), jnp.float32),
            pltpu.VMEM((B, D), jnp.float32),
            pltpu.VMEM((B, D), jnp.float32),
            pltpu.VMEM((B, D), jnp.float32),
            pltpu.VMEM((B, D), jnp.float32),
            pltpu.VMEM((B,), jnp.int32),
            pltpu.VMEM((B,), jnp.int32),
            pltpu.VMEM((B,), jnp.int32),
            pltpu.VMEM((B,), jnp.int32),
            pltpu.VMEM((B,), jnp.int32),
            pltpu.VMEM((B,), jnp.int32),
            pltpu.VMEM((D,), jnp.float32),
            pltpu.VMEM((SUB * 16,), jnp.int32),
            pltpu.VMEM((RP, D), jnp.float32),
            pltpu.VMEM((RP * 16,), jnp.float32),
            pltpu.SMEM((2 * B,), jnp.int32),
            pltpu.SMEM((SUB,), jnp.int32),
            pltpu.SemaphoreType.DMA,
            pltpu.SemaphoreType.DMA,
            pltpu.SemaphoreType.DMA,
            pltpu.SemaphoreType.DMA,
            pltpu.SemaphoreType.DMA,
            pltpu.SemaphoreType.DMA,
            pltpu.SemaphoreType.DMA,
        ],
    )
    return f(lsrc, ldst, leid, cnts, xl, xr, ep, att)


def kernel(x, edge_index, edge_attr, batch, W_pre_x, b_pre_x, W_pre_e, b_pre_e,
           Wl, bl, Wr, br, We, att, conv_b, W_pred, b_pred):
    xl, xr = pl.pallas_call(
        _node_proj_body,
        grid=(N // NT,),
        in_specs=[
            pl.BlockSpec((NT, 128), lambda i: (i, 0)),
            pl.BlockSpec((128, D), lambda i: (0, 0)),
            pl.BlockSpec((D,), lambda i: (0,)),
            pl.BlockSpec((D, D), lambda i: (0, 0)),
            pl.BlockSpec((D,), lambda i: (0,)),
            pl.BlockSpec((D, D), lambda i: (0, 0)),
            pl.BlockSpec((D,), lambda i: (0,)),
        ],
        out_specs=[
            pl.BlockSpec((NT, D), lambda i: (i, 0)),
            pl.BlockSpec((NT, D), lambda i: (i, 0)),
        ],
        out_shape=[
            jax.ShapeDtypeStruct((N, D), jnp.float32),
            jax.ShapeDtypeStruct((N, D), jnp.float32),
        ],
    )(x, W_pre_x, b_pre_x, Wl, bl, Wr, br)

    ep = pl.pallas_call(
        _edge_proj_body,
        grid=(E // ET,),
        in_specs=[
            pl.BlockSpec((ET, 16), lambda i: (i, 0)),
            pl.BlockSpec((16, 512), lambda i: (0, 0)),
            pl.BlockSpec((512,), lambda i: (0,)),
            pl.BlockSpec((512, D), lambda i: (0, 0)),
        ],
        out_specs=pl.BlockSpec((ET, D), lambda i: (i, 0)),
        out_shape=jax.ShapeDtypeStruct((E, D), jnp.float32),
    )(edge_attr, W_pre_e, b_pre_e, We)

    src = edge_index[0]
    dst = edge_index[1]
    lists = _scan_call(src, dst)
    outs, dens = _sc_call(lists, xl, xr, ep, att)

    out_pad = outs
    den_pad = dens.reshape(P, 16)
    sent = jnp.full((P - N,), G, jnp.int32)
    batch_pad = jnp.concatenate([batch, sent]).reshape(1, P)

    x_pool, logits, probs = pl.pallas_call(
        _pool_body,
        in_specs=[
            pl.BlockSpec((P, D), lambda: (0, 0)),
            pl.BlockSpec((P, 16), lambda: (0, 0)),
            pl.BlockSpec((1, P), lambda: (0, 0)),
            pl.BlockSpec((D,), lambda: (0,)),
            pl.BlockSpec((D, 10), lambda: (0, 0)),
            pl.BlockSpec((10,), lambda: (0,)),
        ],
        out_specs=[
            pl.BlockSpec((G, D), lambda: (0, 0)),
            pl.BlockSpec((G, 10), lambda: (0, 0)),
            pl.BlockSpec((G, 10), lambda: (0, 0)),
        ],
        out_shape=[
            jax.ShapeDtypeStruct((G, D), jnp.float32),
            jax.ShapeDtypeStruct((G, 10), jnp.float32),
            jax.ShapeDtypeStruct((G, 10), jnp.float32),
        ],
    )(out_pad, den_pad, batch_pad, conv_b, W_pred, b_pred)
    return (x_pool, logits, probs)
